# baseline (speedup 1.0000x reference)
import functools

import jax
import jax.numpy as jnp
from jax import lax
from jax.experimental import pallas as pl
from jax.experimental.pallas import tpu as pltpu
from jax.experimental.pallas import tpu_sc as plsc

_B, _L, _K = 8, 2048, 48
_MAXREL = 32
_NCLS = 2 * _MAXREL + 1 + 1
_DOUT = 16
_LANES = 16
_NC, _NS = 2, 16
_NW = _NC * _NS
_WPB = _NW // _B
_ROWS = _L // _WPB
_CH = 32
_NCHUNK = _ROWS // _CH
_KV = _K // _LANES
_CHW = _CH * _K * _DOUT


def _sc_body(r_hbm, c_hbm, e_hbm, wt_hbm, b_hbm, out_hbm,
             r_v, c_v, t_v, b_v, e_v, o_v):
    cid = lax.axis_index("c")
    sid = lax.axis_index("s")
    wid = sid * _NC + cid
    bb = wid // _WPB
    l0 = (wid % _WPB) * _ROWS

    pltpu.sync_copy(r_hbm.at[bb], r_v)
    pltpu.sync_copy(c_hbm.at[bb], c_v)
    pltpu.sync_copy(e_hbm.at[bb, pl.ds(l0, _ROWS)], e_v)
    pltpu.sync_copy(wt_hbm, t_v)
    pltpu.sync_copy(b_hbm, b_v)

    bvec = b_v[...]
    for i in range(_NCLS):
        sl = pl.ds(i * _DOUT, _DOUT)
        t_v[sl] = t_v[sl] + bvec

    s_iota = lax.iota(jnp.int32, _LANES) * _DOUT

    for ci in range(_NCHUNK):
        @pl.loop(0, _CH)
        def _row(rr):
            row = ci * _CH + rr
            anchor = jnp.full((_LANES,), l0 + row, dtype=jnp.int32)
            ra = plsc.load_gather(r_v, [anchor])
            ca = plsc.load_gather(c_v, [anchor])
            for kk in range(_KV):
                e = e_v[row, pl.ds(kk * _LANES, _LANES)]
                rj = plsc.load_gather(r_v, [e])
                cj = plsc.load_gather(c_v, [e])
                dd = jnp.clip(ra - rj, -_MAXREL, _MAXREL) + _MAXREL
                dd = jnp.where(ca == cj, dd, _NCLS - 1)
                g0 = dd * _DOUT
                s0 = rr * (_K * _DOUT) + kk * (_LANES * _DOUT) + s_iota
                for j in range(_DOUT):
                    col = plsc.load_gather(t_v, [g0 + j])
                    plsc.store_scatter(o_v, [s0 + j], col)

        base = (bb * _L + l0 + ci * _CH) * (_K * _DOUT)
        pltpu.sync_copy(o_v, out_hbm.at[pl.ds(base, _CHW)])


_sc_kernel = functools.partial(
    pl.kernel,
    out_type=jax.ShapeDtypeStruct((_B * _L * _K * _DOUT,), jnp.float32),
    mesh=plsc.VectorSubcoreMesh(core_axis_name="c", subcore_axis_name="s",
                                num_cores=_NC, num_subcores=_NS),
    scratch_types=[
        pltpu.VMEM((_L,), jnp.int32),
        pltpu.VMEM((_L,), jnp.int32),
        pltpu.VMEM((_NCLS * _DOUT,), jnp.float32),
        pltpu.VMEM((_DOUT,), jnp.float32),
        pltpu.VMEM((_ROWS, _K), jnp.int32),
        pltpu.VMEM((_CHW,), jnp.float32),
    ],
    compiler_params=pltpu.CompilerParams(needs_layout_passes=False),
)(_sc_body)


def kernel(R_idx, chain_labels, E_idx, W, b):
    wt_flat = jnp.transpose(W).reshape(-1)
    out = _sc_kernel(R_idx, chain_labels, E_idx, wt_flat, b)
    return out.reshape(_B, _L, _K, _DOUT)


# speedup vs baseline: 3.6948x; 3.6948x over previous
import functools

import jax
import jax.numpy as jnp
from jax import lax
from jax.experimental import pallas as pl
from jax.experimental.pallas import tpu as pltpu
from jax.experimental.pallas import tpu_sc as plsc

_B, _L, _K = 8, 2048, 48
_MAXREL = 32
_NCLS = 2 * _MAXREL + 1 + 1
_DOUT = 16
_LANES = 16
_NC, _NS = 2, 16
_NW = _NC * _NS
_WPB = _NW // _B
_LW = _L // _WPB
_LCH = _LW // _LANES


def _sc_body(r_hbm, c_hbm, e_hbm, wt_hbm, b_hbm, out_hbm,
             r_v, c_v, t_v, b_v, e_v, o_v, sem):
    cid = lax.axis_index("c")
    sid = lax.axis_index("s")
    wid = sid * _NC + cid
    bb = wid // _WPB
    l0 = (wid % _WPB) * _LW

    pltpu.sync_copy(r_hbm.at[bb], r_v)
    pltpu.sync_copy(c_hbm.at[bb], c_v)
    pltpu.sync_copy(e_hbm.at[bb, :, pl.ds(l0, _LW)], e_v)
    pltpu.sync_copy(wt_hbm, t_v)
    pltpu.sync_copy(b_hbm, b_v)

    bvec = b_v[...]
    for i in range(_NCLS):
        sl = pl.ds(i * _DOUT, _DOUT)
        t_v[sl] = t_v[sl] + bvec

    def compute_k(k, slot):
        @pl.loop(0, _LCH)
        def _lc(lc):
            l16 = lc * _LANES
            e = e_v[k, pl.ds(l16, _LANES)]
            ra = r_v[pl.ds(l0 + l16, _LANES)]
            ca = c_v[pl.ds(l0 + l16, _LANES)]
            rj = plsc.load_gather(r_v, [e])
            cj = plsc.load_gather(c_v, [e])
            dd = jnp.clip(ra - rj, -_MAXREL, _MAXREL) + _MAXREL
            dd = jnp.where(ca == cj, dd, _NCLS - 1)
            g0 = dd * _DOUT
            for j in range(_DOUT):
                o_v[slot, j, pl.ds(l16, _LANES)] = plsc.load_gather(t_v, [g0 + j])

    compute_k(0, 0)
    @pl.loop(1, _K)
    def _k(k):
        prev = (k - 1) % 2
        cur = k % 2
        cp = pltpu.async_copy(o_v.at[prev], out_hbm.at[bb, k - 1, :, pl.ds(l0, _LW)], sem)
        compute_k(k, cur)
        cp.wait()
    pltpu.sync_copy(o_v.at[(_K - 1) % 2], out_hbm.at[bb, _K - 1, :, pl.ds(l0, _LW)])


_sc_kernel = functools.partial(
    pl.kernel,
    out_type=jax.ShapeDtypeStruct((_B, _K, _DOUT, _L), jnp.float32),
    mesh=plsc.VectorSubcoreMesh(core_axis_name="c", subcore_axis_name="s",
                                num_cores=_NC, num_subcores=_NS),
    scratch_types=[
        pltpu.VMEM((_L,), jnp.int32),
        pltpu.VMEM((_L,), jnp.int32),
        pltpu.VMEM((_NCLS * _DOUT,), jnp.float32),
        pltpu.VMEM((_DOUT,), jnp.float32),
        pltpu.VMEM((_K, _LW), jnp.int32),
        pltpu.VMEM((2, _DOUT, _LW), jnp.float32),
        pltpu.SemaphoreType.DMA,
    ],
    compiler_params=pltpu.CompilerParams(needs_layout_passes=False),
)(_sc_body)


def kernel(R_idx, chain_labels, E_idx, W, b):
    wt_flat = jnp.transpose(W).reshape(-1)
    e_t = jnp.transpose(E_idx, (0, 2, 1))
    out = _sc_kernel(R_idx, chain_labels, e_t, wt_flat, b)
    return jnp.transpose(out, (0, 3, 1, 2))


# speedup vs baseline: 9.5869x; 2.5947x over previous
import functools

import jax
import jax.numpy as jnp
from jax import lax
from jax.experimental import pallas as pl
from jax.experimental.pallas import tpu as pltpu
from jax.experimental.pallas import tpu_sc as plsc

_B, _L, _K = 8, 2048, 48
_MAXREL = 32
_NCLS = 2 * _MAXREL + 1 + 1
_DOUT = 16
_LANES = 16
_NC, _NS = 2, 16
_NW = _NC * _NS
_WPB = _NW // _B
_LW = _L // _WPB
_LCH = _LW // _LANES


def _sc_body(r_hbm, c_hbm, e_hbm, wt_hbm, b_hbm, out_hbm,
             r_v, c_v, t_v, b_v, e_v, o_v, sem):
    cid = lax.axis_index("c")
    sid = lax.axis_index("s")
    wid = sid * _NC + cid
    bb = wid // _WPB
    l0 = (wid % _WPB) * _LW

    pltpu.sync_copy(r_hbm.at[bb], r_v)
    pltpu.sync_copy(c_hbm.at[bb], c_v)
    pltpu.sync_copy(e_hbm.at[bb, :, pl.ds(l0, _LW)], e_v)
    pltpu.sync_copy(wt_hbm, t_v)
    pltpu.sync_copy(b_hbm, b_v)

    bvec = b_v[...]
    for i in range(_NCLS):
        sl = pl.ds(i * _DOUT, _DOUT)
        t_v[sl] = t_v[sl] + bvec

    def compute_k(k, slot):
        @plsc.parallel_loop(0, _LCH, unroll=2)
        def _lc(lc):
            l16 = lc * _LANES
            e = e_v[k, pl.ds(l16, _LANES)]
            ra = r_v[pl.ds(l0 + l16, _LANES)]
            ca = c_v[pl.ds(l0 + l16, _LANES)]
            rj = plsc.load_gather(r_v, [e])
            cj = plsc.load_gather(c_v, [e])
            dd = jnp.clip(ra - rj, -_MAXREL, _MAXREL) + _MAXREL
            dd = jnp.where(ca == cj, dd, _NCLS - 1)
            g0 = dd * _DOUT
            for j in range(_DOUT):
                o_v[slot, j, pl.ds(l16, _LANES)] = plsc.load_gather(t_v, [g0 + j])

    compute_k(0, 0)
    @pl.loop(1, _K)
    def _k(k):
        prev = (k - 1) % 2
        cur = k % 2
        cp = pltpu.async_copy(o_v.at[prev], out_hbm.at[bb, k - 1, :, pl.ds(l0, _LW)], sem)
        compute_k(k, cur)
        cp.wait()
    pltpu.sync_copy(o_v.at[(_K - 1) % 2], out_hbm.at[bb, _K - 1, :, pl.ds(l0, _LW)])


_sc_kernel = functools.partial(
    pl.kernel,
    out_type=jax.ShapeDtypeStruct((_B, _K, _DOUT, _L), jnp.float32),
    mesh=plsc.VectorSubcoreMesh(core_axis_name="c", subcore_axis_name="s",
                                num_cores=_NC, num_subcores=_NS),
    scratch_types=[
        pltpu.VMEM((_L,), jnp.int32),
        pltpu.VMEM((_L,), jnp.int32),
        pltpu.VMEM((_NCLS * _DOUT,), jnp.float32),
        pltpu.VMEM((_DOUT,), jnp.float32),
        pltpu.VMEM((_K, _LW), jnp.int32),
        pltpu.VMEM((2, _DOUT, _LW), jnp.float32),
        pltpu.SemaphoreType.DMA,
    ],
    compiler_params=pltpu.CompilerParams(needs_layout_passes=False),
)(_sc_body)


def kernel(R_idx, chain_labels, E_idx, W, b):
    wt_flat = jnp.transpose(W).reshape(-1)
    e_t = jnp.transpose(E_idx, (0, 2, 1))
    out = _sc_kernel(R_idx, chain_labels, e_t, wt_flat, b)
    return jnp.transpose(out, (0, 3, 1, 2))
